# baseline (speedup 1.0000x reference)
import functools

import jax
import jax.numpy as jnp
from jax import lax
from jax.experimental import pallas as pl
from jax.experimental.pallas import tpu as pltpu
from jax.experimental.pallas import tpu_sc as plsc

T = 2048
D = 768
E = 8
H = 4 * D
BLK = 256
NB = 15
P = NB * BLK
CAP = 2048
EPS = 1e-6

NUM_CORES = 2
NUM_SUBCORES = 16
NW = NUM_CORES * NUM_SUBCORES
CHUNK = T // NW
LANES = 16
CLANE = 128


def _route_body(x_ref, wg_ref, bg_ref, dst_ref, coef_ref, seg_ref, loss_ref):
    x = x_ref[...]
    wg = wg_ref[...]
    logits = lax.dot_general(x, wg, (((1,), (1,)), ((), ())),
                             preferred_element_type=jnp.float32)
    logits = logits + bg_ref[...]
    m = jnp.max(logits, axis=1, keepdims=True)
    ex = jnp.exp(logits - m)
    gate = ex / jnp.sum(ex, axis=1, keepdims=True)

    p = jnp.max(gate, axis=1, keepdims=True)
    colid = lax.broadcasted_iota(jnp.int32, (T, E), 1)
    e_idx = jnp.min(jnp.where(gate == p, colid, E), axis=1, keepdims=True)
    onehot = (colid == e_idx).astype(jnp.float32)

    msum = jnp.sum(gate * onehot, axis=0, keepdims=True)
    denom = msum + EPS
    coef = p * jnp.sum(onehot / denom, axis=1, keepdims=True) * CAP
    coef_ref[...] = jnp.broadcast_to(coef, (T, CLANE))

    load = msum / denom * CAP
    importance = jnp.sum(load) / T
    loss_ref[...] = jnp.broadcast_to(jnp.mean((load - importance) ** 2), (1, 1))

    G = 16
    S = T // G
    oh3 = onehot.reshape(G, S, E)
    gs = jnp.sum(oh3, axis=1)
    trig = jnp.where(lax.broadcasted_iota(jnp.int32, (G, G), 0)
                     < lax.broadcasted_iota(jnp.int32, (G, G), 1), 1.0, 0.0)
    gpre = lax.dot_general(trig, gs, (((0,), (0,)), ((), ())),
                           preferred_element_type=jnp.float32)
    tris = jnp.where(lax.broadcasted_iota(jnp.int32, (S, S), 0)
                     > lax.broadcasted_iota(jnp.int32, (S, S), 1),
                     1.0, 0.0).astype(jnp.bfloat16)
    parts = []
    for g in range(G):
        rk_g = lax.dot_general(tris, oh3[g].astype(jnp.bfloat16),
                               (((1,), (0,)), ((), ())),
                               preferred_element_type=jnp.float32)
        parts.append(rk_g + gpre[g:g + 1, :])
    rank_mat = jnp.concatenate(parts, axis=0)
    rank = jnp.sum(rank_mat * onehot, axis=1, keepdims=True)

    counts = jnp.sum(onehot, axis=0, keepdims=True)
    nb_e = jnp.floor((counts + (BLK - 1)) * (1.0 / BLK))
    tri8 = jnp.where(lax.broadcasted_iota(jnp.int32, (E, E), 0)
                     < lax.broadcasted_iota(jnp.int32, (E, E), 1), 1.0, 0.0)
    pad_blk = lax.dot_general(nb_e, tri8, (((1,), (0,)), ((), ())),
                              preferred_element_type=jnp.float32)
    pad_off = pad_blk * BLK
    dst = jnp.sum(onehot * pad_off, axis=1, keepdims=True) + rank
    dst_ref[...] = dst.astype(jnp.int32)

    seg_ref[...] = jnp.concatenate([pad_blk, nb_e], axis=0).astype(jnp.int32)


_route = pl.pallas_call(
    _route_body,
    out_shape=[
        jax.ShapeDtypeStruct((T, 1), jnp.int32),
        jax.ShapeDtypeStruct((T, CLANE), jnp.float32),
        jax.ShapeDtypeStruct((2, E), jnp.int32),
        jax.ShapeDtypeStruct((1, 1), jnp.float32),
    ],
)


@functools.cache
def _sc_kernels():
    mesh = plsc.VectorSubcoreMesh(core_axis_name="c", subcore_axis_name="s",
                                  num_cores=NUM_CORES, num_subcores=NUM_SUBCORES)

    @functools.partial(
        pl.kernel, mesh=mesh,
        out_type=[
            jax.ShapeDtypeStruct((P, D), jnp.float32),
            jax.ShapeDtypeStruct((P, CLANE), jnp.float32),
        ],
        scratch_types=[
            pltpu.VMEM((CHUNK,), jnp.int32),
            pltpu.VMEM((CHUNK, D), jnp.float32),
            pltpu.VMEM((CHUNK, CLANE), jnp.float32),
            pltpu.SemaphoreType.DMA,
            pltpu.SemaphoreType.DMA,
        ],
    )
    def _scatter_rows(x_hbm, dst_hbm, coef_hbm, xp_hbm, cp_hbm,
                      idx_v, rows_v, coefr_v, sem, sem2):
        wid = lax.axis_index("s") * NUM_CORES + lax.axis_index("c")
        base = wid * CHUNK
        pltpu.sync_copy(dst_hbm.at[pl.ds(base, CHUNK)], idx_v)
        pltpu.sync_copy(x_hbm.at[pl.ds(base, CHUNK)], rows_v)
        pltpu.sync_copy(coef_hbm.at[pl.ds(base, CHUNK)], coefr_v)
        c1 = pltpu.async_copy(rows_v, xp_hbm.at[idx_v], sem)
        c2 = pltpu.async_copy(coefr_v, cp_hbm.at[idx_v], sem2)
        c1.wait()
        c2.wait()

    @functools.partial(
        pl.kernel, mesh=mesh,
        out_type=jax.ShapeDtypeStruct((T, D), jnp.float32),
        scratch_types=[
            pltpu.VMEM((CHUNK,), jnp.int32),
            pltpu.VMEM((CHUNK, D), jnp.float32),
            pltpu.SemaphoreType.DMA,
        ],
    )
    def _gather_rows(yp_hbm, dst_hbm, out_hbm, idx_v, rows_v, sem):
        wid = lax.axis_index("s") * NUM_CORES + lax.axis_index("c")
        base = wid * CHUNK
        pltpu.sync_copy(dst_hbm.at[pl.ds(base, CHUNK)], idx_v)
        pltpu.async_copy(yp_hbm.at[idx_v], rows_v, sem).wait()
        pltpu.sync_copy(rows_v, out_hbm.at[pl.ds(base, CHUNK)])

    return _scatter_rows, _gather_rows


H2 = H // 2
H4 = H // 4


def _ffn_body(info_ref, xp_ref, cp_ref, w1a_ref, w1b_ref, b1a_ref, b1b_ref,
              w2a_ref, w2b_ref, b2_ref, out_ref):
    e = pl.program_id(0)
    j = pl.program_id(1)
    start = info_ref[0, e]
    nb = info_ref[1, e]

    def run(r0, rows):
        xb = xp_ref[pl.ds(r0, rows), :]
        y = None
        for w1q, b1q, w2q in ((w1a_ref, b1a_ref, w2a_ref),
                              (w1b_ref, b1b_ref, w2b_ref)):
            h = lax.dot_general(xb, w1q[...], (((1,), (1,)), ((), ())),
                                preferred_element_type=jnp.float32)
            h = h + b1q[0]
            h = h * jax.nn.sigmoid(1.702 * h)
            yq = lax.dot_general(h, w2q[...], (((1,), (1,)), ((), ())),
                                 preferred_element_type=jnp.float32)
            y = yq if y is None else y + yq

        @pl.when(j == 0)
        def _():
            out_ref[pl.ds(r0, rows), :] = y

        @pl.when(j == 1)
        def _():
            acc = out_ref[pl.ds(r0, rows), :] + y + b2_ref[0]
            out_ref[pl.ds(r0, rows), :] = acc * cp_ref[pl.ds(r0, rows), 0:1]

    n2 = nb // 2

    def body(k, carry):
        run(start * BLK + k * (2 * BLK), 2 * BLK)
        return carry

    lax.fori_loop(0, n2, body, 0)

    @pl.when(nb % 2 == 1)
    def _():
        run((start + nb - 1) * BLK, BLK)


_ffn = pl.pallas_call(
    _ffn_body,
    grid_spec=pltpu.PrefetchScalarGridSpec(
        num_scalar_prefetch=1,
        grid=(E, 2),
        in_specs=[
            pl.BlockSpec((P, D), lambda e, j, info: (0, 0)),
            pl.BlockSpec((P, CLANE), lambda e, j, info: (0, 0)),
            pl.BlockSpec((H4, D), lambda e, j, info: (4 * e + 2 * j, 0)),
            pl.BlockSpec((H4, D), lambda e, j, info: (4 * e + 2 * j + 1, 0)),
            pl.BlockSpec((1, 1, H4), lambda e, j, info: (e, 0, 2 * j)),
            pl.BlockSpec((1, 1, H4), lambda e, j, info: (e, 0, 2 * j + 1)),
            pl.BlockSpec((D, H4), lambda e, j, info: (e, 2 * j)),
            pl.BlockSpec((D, H4), lambda e, j, info: (e, 2 * j + 1)),
            pl.BlockSpec((1, 1, D), lambda e, j, info: (e, 0, 0)),
        ],
        out_specs=pl.BlockSpec((P, D), lambda e, j, info: (0, 0)),
    ),
    out_shape=jax.ShapeDtypeStruct((P, D), jnp.float32),
    compiler_params=pltpu.CompilerParams(
        vmem_limit_bytes=64 * 1024 * 1024,
    ),
)


def kernel(x, wg, bg, W1, b1, W2, b2):
    scatter_rows, gather_rows = _sc_kernels()
    dst2, coef2, info, loss = _route(x, wg, bg.reshape(1, E))
    dst = dst2.reshape(T)
    xp, cp = scatter_rows(x, dst, coef2)
    w1r = W1.reshape(E * H, D)
    w2r = W2.reshape(E * D, H)
    b1r = b1.reshape(E, 1, H)
    yp = _ffn(info, xp, cp, w1r, w1r, b1r, b1r, w2r, w2r, b2.reshape(E, 1, D))
    out = gather_rows(yp, dst)
    return out, loss.reshape(())


# speedup vs baseline: 1.0322x; 1.0322x over previous
import functools

import jax
import jax.numpy as jnp
from jax import lax
from jax.experimental import pallas as pl
from jax.experimental.pallas import tpu as pltpu
from jax.experimental.pallas import tpu_sc as plsc

T = 2048
D = 768
E = 8
H = 4 * D
BLK = 256
NB = 15
P = NB * BLK
CAP = 2048
EPS = 1e-6

NUM_CORES = 2
NUM_SUBCORES = 16
NW = NUM_CORES * NUM_SUBCORES
CHUNK = T // NW
LANES = 16
CLANE = 128


def _route_body(x_ref, wg_ref, bg_ref, dst_ref, coef_ref, seg_ref, loss_ref):
    x = x_ref[...]
    wg = wg_ref[...]
    logits = lax.dot_general(x, wg, (((1,), (1,)), ((), ())),
                             preferred_element_type=jnp.float32)
    logits = logits + bg_ref[...]
    m = jnp.max(logits, axis=1, keepdims=True)
    ex = jnp.exp(logits - m)
    gate = ex / jnp.sum(ex, axis=1, keepdims=True)

    p = jnp.max(gate, axis=1, keepdims=True)
    colid = lax.broadcasted_iota(jnp.int32, (T, E), 1)
    e_idx = jnp.min(jnp.where(gate == p, colid, E), axis=1, keepdims=True)
    onehot = (colid == e_idx).astype(jnp.float32)

    msum = jnp.sum(gate * onehot, axis=0, keepdims=True)
    denom = msum + EPS
    coef = p * jnp.sum(onehot / denom, axis=1, keepdims=True) * CAP
    coef_ref[...] = jnp.broadcast_to(coef, (T, CLANE))

    load = msum / denom * CAP
    importance = jnp.sum(load) / T
    loss_ref[...] = jnp.broadcast_to(jnp.mean((load - importance) ** 2), (1, 1))

    G = 16
    S = T // G
    oh3 = onehot.reshape(G, S, E)
    gs = jnp.sum(oh3, axis=1)
    trig = jnp.where(lax.broadcasted_iota(jnp.int32, (G, G), 0)
                     < lax.broadcasted_iota(jnp.int32, (G, G), 1), 1.0, 0.0)
    gpre = lax.dot_general(trig, gs, (((0,), (0,)), ((), ())),
                           preferred_element_type=jnp.float32)
    tris = jnp.where(lax.broadcasted_iota(jnp.int32, (S, S), 0)
                     > lax.broadcasted_iota(jnp.int32, (S, S), 1),
                     1.0, 0.0).astype(jnp.bfloat16)
    parts = []
    for g in range(G):
        rk_g = lax.dot_general(tris, oh3[g].astype(jnp.bfloat16),
                               (((1,), (0,)), ((), ())),
                               preferred_element_type=jnp.float32)
        parts.append(rk_g + gpre[g:g + 1, :])
    rank_mat = jnp.concatenate(parts, axis=0)
    rank = jnp.sum(rank_mat * onehot, axis=1, keepdims=True)

    counts = jnp.sum(onehot, axis=0, keepdims=True)
    nb_e = jnp.floor((counts + (BLK - 1)) * (1.0 / BLK))
    tri8 = jnp.where(lax.broadcasted_iota(jnp.int32, (E, E), 0)
                     < lax.broadcasted_iota(jnp.int32, (E, E), 1), 1.0, 0.0)
    pad_blk = lax.dot_general(nb_e, tri8, (((1,), (0,)), ((), ())),
                              preferred_element_type=jnp.float32)
    pad_off = pad_blk * BLK
    dst = jnp.sum(onehot * pad_off, axis=1, keepdims=True) + rank
    dst_ref[...] = dst.astype(jnp.int32)

    seg_ref[...] = jnp.concatenate([pad_blk, nb_e], axis=0).astype(jnp.int32)


_route = pl.pallas_call(
    _route_body,
    out_shape=[
        jax.ShapeDtypeStruct((T, 1), jnp.int32),
        jax.ShapeDtypeStruct((T, CLANE), jnp.float32),
        jax.ShapeDtypeStruct((2, E), jnp.int32),
        jax.ShapeDtypeStruct((1, 1), jnp.float32),
    ],
)


@functools.cache
def _sc_kernels():
    mesh = plsc.VectorSubcoreMesh(core_axis_name="c", subcore_axis_name="s",
                                  num_cores=NUM_CORES, num_subcores=NUM_SUBCORES)

    @functools.partial(
        pl.kernel, mesh=mesh,
        out_type=[
            jax.ShapeDtypeStruct((P, D), jnp.float32),
            jax.ShapeDtypeStruct((P, CLANE), jnp.float32),
        ],
        scratch_types=[
            pltpu.VMEM((CHUNK,), jnp.int32),
            pltpu.VMEM((CHUNK, D), jnp.float32),
            pltpu.VMEM((CHUNK, CLANE), jnp.float32),
            pltpu.SemaphoreType.DMA,
            pltpu.SemaphoreType.DMA,
        ],
    )
    def _scatter_rows(x_hbm, dst_hbm, coef_hbm, xp_hbm, cp_hbm,
                      idx_v, rows_v, coefr_v, sem, sem2):
        wid = lax.axis_index("s") * NUM_CORES + lax.axis_index("c")
        base = wid * CHUNK
        pltpu.sync_copy(dst_hbm.at[pl.ds(base, CHUNK)], idx_v)
        pltpu.sync_copy(x_hbm.at[pl.ds(base, CHUNK)], rows_v)
        pltpu.sync_copy(coef_hbm.at[pl.ds(base, CHUNK)], coefr_v)
        c1 = pltpu.async_copy(rows_v, xp_hbm.at[idx_v], sem)
        c2 = pltpu.async_copy(coefr_v, cp_hbm.at[idx_v], sem2)
        c1.wait()
        c2.wait()

    @functools.partial(
        pl.kernel, mesh=mesh,
        out_type=jax.ShapeDtypeStruct((T, D), jnp.float32),
        scratch_types=[
            pltpu.VMEM((CHUNK,), jnp.int32),
            pltpu.VMEM((CHUNK, D), jnp.float32),
            pltpu.SemaphoreType.DMA,
        ],
    )
    def _gather_rows(yp_hbm, dst_hbm, out_hbm, idx_v, rows_v, sem):
        wid = lax.axis_index("s") * NUM_CORES + lax.axis_index("c")
        base = wid * CHUNK
        pltpu.sync_copy(dst_hbm.at[pl.ds(base, CHUNK)], idx_v)
        pltpu.async_copy(yp_hbm.at[idx_v], rows_v, sem).wait()
        pltpu.sync_copy(rows_v, out_hbm.at[pl.ds(base, CHUNK)])

    return _scatter_rows, _gather_rows


H2 = H // 2


def _ffn_body(info_ref, xp_ref, cp_ref, w1_ref, b1_ref, w2_ref, b2_ref, out_ref):
    e = pl.program_id(0)
    j = pl.program_id(1)
    start = info_ref[0, e]
    nb = info_ref[1, e]

    def run(r0, rows):
        xb = xp_ref[pl.ds(r0, rows), :]
        h = lax.dot_general(xb, w1_ref[...], (((1,), (1,)), ((), ())),
                            preferred_element_type=jnp.float32)
        h = h + b1_ref[0]
        h = h * jax.nn.sigmoid(1.702 * h)
        y = lax.dot_general(h, w2_ref[...], (((1,), (1,)), ((), ())),
                            preferred_element_type=jnp.float32)

        @pl.when(j == 0)
        def _():
            out_ref[pl.ds(r0, rows), :] = y

        @pl.when(j == 1)
        def _():
            acc = out_ref[pl.ds(r0, rows), :] + y + b2_ref[0]
            out_ref[pl.ds(r0, rows), :] = acc * cp_ref[pl.ds(r0, rows), 0:1]

    n2 = nb // 2

    def body(k, carry):
        run(start * BLK + k * (2 * BLK), 2 * BLK)
        return carry

    lax.fori_loop(0, n2, body, 0)

    @pl.when(nb % 2 == 1)
    def _():
        run((start + nb - 1) * BLK, BLK)


_ffn = pl.pallas_call(
    _ffn_body,
    grid_spec=pltpu.PrefetchScalarGridSpec(
        num_scalar_prefetch=1,
        grid=(E, 2),
        in_specs=[
            pl.BlockSpec((P, D), lambda e, j, info: (0, 0)),
            pl.BlockSpec((P, CLANE), lambda e, j, info: (0, 0)),
            pl.BlockSpec((H2, D), lambda e, j, info: (2 * e + j, 0)),
            pl.BlockSpec((1, 1, H2), lambda e, j, info: (e, 0, j)),
            pl.BlockSpec((D, H2), lambda e, j, info: (e, j)),
            pl.BlockSpec((1, 1, D), lambda e, j, info: (e, 0, 0)),
        ],
        out_specs=pl.BlockSpec((P, D), lambda e, j, info: (0, 0)),
    ),
    out_shape=jax.ShapeDtypeStruct((P, D), jnp.float32),
    compiler_params=pltpu.CompilerParams(
        vmem_limit_bytes=64 * 1024 * 1024,
    ),
)


def kernel(x, wg, bg, W1, b1, W2, b2):
    scatter_rows, gather_rows = _sc_kernels()
    dst2, coef2, info, loss = _route(x, wg, bg.reshape(1, E))
    dst = dst2.reshape(T)
    xp, cp = scatter_rows(x, dst, coef2)
    yp = _ffn(info, xp, cp, W1.reshape(E * H, D), b1.reshape(E, 1, H),
              W2.reshape(E * D, H), b2.reshape(E, 1, D))
    out = gather_rows(yp, dst)
    return out, loss.reshape(())
